# Initial kernel scaffold; baseline (speedup 1.0000x reference)
import jax
import jax.numpy as jnp
from jax.experimental import pallas as pl


def kernel(x, base_W, bias, qweight0, qweight1, scales0, scales1, indices):
    raise NotImplementedError("write your pallas kernel here")


# speedup vs baseline: 1.3034x; 1.3034x over previous
import functools

import jax
import jax.numpy as jnp
from jax import lax
from jax.experimental import pallas as pl
from jax.experimental.pallas import tpu as pltpu

G = 8
BT = 256
BN = 512


def _grouped_body(rows, grps, firsts, starts, ends,
                  x_ref, qw0, qw1, s0, s1, bw, bias_ref, out_ref,
                  xbf, wbf, bwbf):
    c = pl.program_id(0)
    w = pl.program_id(1)
    g = grps[w]
    r = rows[w]
    first = firsts[w]
    start = starts[w]
    end = ends[w]

    @pl.when((c == 0) & (w == 0))
    def _():
        xbf[...] = x_ref[...].astype(jnp.bfloat16)

    @pl.when(w == 0)
    def _():
        bwbf[...] = bw[...].astype(jnp.bfloat16)

    prev_g = grps[jnp.maximum(w - 1, 0)]
    new_w = (w == 0) | (g != prev_g)

    @pl.when(new_w & (c < 4))
    def _():
        wbf[...] = (qw0[0] - 8).astype(jnp.bfloat16)

    @pl.when(new_w & (c >= 4))
    def _():
        wbf[...] = (qw1[0] - 8).astype(jnp.bfloat16)

    scale = jnp.where(c < 4, s0[0, 0, 0, :], s1[0, 0, 0, :])

    xb = xbf[pl.ds(r * BT, BT), :]
    delta = lax.dot_general(xb, wbf[...], (((1,), (1,)), ((), ())),
                            preferred_element_type=jnp.float32)
    delta = delta * scale[None, :]
    row_ids = r * BT + lax.broadcasted_iota(jnp.int32, (BT, 1), 0)
    mask = (row_ids >= start) & (row_ids < end)
    contrib = jnp.where(mask, delta, 0.0)

    @pl.when(first == 1)
    def _():
        base = lax.dot_general(xb, bwbf[...], (((1,), (1,)), ((), ())),
                               preferred_element_type=jnp.float32)
        out_ref[...] = base + bias_ref[0] + contrib

    @pl.when(first == 0)
    def _():
        out_ref[...] += contrib


def _routing_metadata(indices, T, nt):
    W = nt + G - 1
    sizes = jnp.sum((indices[:, None] == jnp.arange(G)[None, :]).astype(jnp.int32),
                    axis=0)
    off = jnp.concatenate([jnp.zeros(1, jnp.int32), jnp.cumsum(sizes)])
    start_t = off[:-1] // BT
    end_t = jnp.where(sizes > 0, (off[1:] - 1) // BT, start_t - 1)
    tiles = jnp.maximum(end_t - start_t + 1, 0)
    cum = jnp.cumsum(tiles)
    wids = jnp.arange(W, dtype=jnp.int32)
    gid = jnp.searchsorted(cum, wids, side='right').astype(jnp.int32)
    gid_c = jnp.minimum(gid, G - 1)
    prev_cum = jnp.where(gid_c > 0, cum[gid_c - 1], 0)
    rid = start_t[gid_c] + (wids - prev_cum)
    valid = wids < cum[-1]
    rid = jnp.where(valid, rid, nt - 1).astype(jnp.int32)
    gcl = jnp.where(valid, gid_c, G - 1).astype(jnp.int32)
    st = jnp.where(valid, off[gcl], 0).astype(jnp.int32)
    en = jnp.where(valid, off[gcl + 1], 0).astype(jnp.int32)
    first = jnp.concatenate([jnp.ones(1, jnp.int32),
                             (rid[1:] != rid[:-1]).astype(jnp.int32)])
    return rid, gcl, first, st, en


@jax.jit
def kernel(x, base_W, bias, qweight0, qweight1, scales0, scales1, indices):
    T, D = x.shape
    NOUT = base_W.shape[0]
    SL = NOUT // 2
    nt = T // BT
    W = nt + G - 1
    nc = NOUT // BN
    nc_s = SL // BN

    rid, gcl, first, st, en = _routing_metadata(indices, T, nt)
    perm = jnp.argsort(indices)
    inv = jnp.argsort(perm)
    x_s = jnp.take(x, perm, axis=0)

    s0r = scales0.reshape(G, nc_s, 1, BN)
    s1r = scales1.reshape(G, nc_s, 1, BN)
    bias_r = bias.reshape(nc, 1, BN)

    grid_spec = pltpu.PrefetchScalarGridSpec(
        num_scalar_prefetch=5,
        grid=(nc, W),
        in_specs=[
            pl.BlockSpec((T, D), lambda c, w, *s: (0, 0)),
            pl.BlockSpec((1, BN, D),
                         lambda c, w, rows, grps, *s: (
                             jnp.where(c < nc_s, grps[w], 0),
                             jnp.where(c < nc_s, c, 0), 0)),
            pl.BlockSpec((1, BN, D),
                         lambda c, w, rows, grps, *s: (
                             jnp.where(c >= nc_s, grps[w], 0),
                             jnp.where(c >= nc_s, c - nc_s, 0), 0)),
            pl.BlockSpec((1, 1, 1, BN),
                         lambda c, w, rows, grps, *s: (
                             jnp.where(c < nc_s, grps[w], 0),
                             jnp.where(c < nc_s, c, 0), 0, 0)),
            pl.BlockSpec((1, 1, 1, BN),
                         lambda c, w, rows, grps, *s: (
                             jnp.where(c >= nc_s, grps[w], 0),
                             jnp.where(c >= nc_s, c - nc_s, 0), 0, 0)),
            pl.BlockSpec((BN, D), lambda c, w, *s: (c, 0)),
            pl.BlockSpec((1, 1, BN), lambda c, w, *s: (c, 0, 0)),
        ],
        out_specs=pl.BlockSpec((BT, BN), lambda c, w, rows, *s: (rows[w], c)),
        scratch_shapes=[
            pltpu.VMEM((T, D), jnp.bfloat16),
            pltpu.VMEM((BN, D), jnp.bfloat16),
            pltpu.VMEM((BN, D), jnp.bfloat16),
        ],
    )

    out_s = pl.pallas_call(
        _grouped_body,
        grid_spec=grid_spec,
        out_shape=jax.ShapeDtypeStruct((T, NOUT), jnp.float32),
        compiler_params=pltpu.CompilerParams(
            dimension_semantics=("arbitrary", "arbitrary")),
    )(rid, gcl, first, st, en,
      x_s, qweight0, qweight1, s0r, s1r, base_W, bias_r)

    return jnp.take(out_s, inv, axis=0)


# speedup vs baseline: 1.5031x; 1.1532x over previous
import functools

import jax
import jax.numpy as jnp
from jax import lax
from jax.experimental import pallas as pl
from jax.experimental.pallas import tpu as pltpu

G = 8
BT = 256
BN = 1024


def _grouped_body(rows, grps, firsts, starts, ends,
                  x_ref, qw0, qw1, s0, s1, bw, bias_ref, out_ref,
                  xbf, wbf):
    c = pl.program_id(0)
    w = pl.program_id(1)
    g = grps[w]
    r = rows[w]
    first = firsts[w]
    start = starts[w]
    end = ends[w]
    nc_s = pl.num_programs(0) // 2

    @pl.when((c == 0) & (w == 0))
    def _():
        xbf[...] = x_ref[...].astype(jnp.bfloat16)

    prev_g = grps[jnp.maximum(w - 1, 0)]
    new_w = (w == 0) | (g != prev_g)

    @pl.when(new_w & (c < nc_s))
    def _():
        scale = s0[0, 0, 0, :]
        wbf[...] = (bw[...] + scale[:, None] *
                    (qw0[0] - 8).astype(jnp.float32)).astype(jnp.bfloat16)

    @pl.when(new_w & (c >= nc_s))
    def _():
        scale = s1[0, 0, 0, :]
        wbf[...] = (bw[...] + scale[:, None] *
                    (qw1[0] - 8).astype(jnp.float32)).astype(jnp.bfloat16)

    row_ids = r * BT + lax.broadcasted_iota(jnp.int32, (BT, 1), 0)
    mask = (row_ids >= start) & (row_ids < end)
    xb = jnp.where(mask, xbf[pl.ds(r * BT, BT), :], jnp.bfloat16(0))
    contrib = lax.dot_general(xb, wbf[...], (((1,), (1,)), ((), ())),
                              preferred_element_type=jnp.float32)

    @pl.when(first == 1)
    def _():
        out_ref[...] = contrib + bias_ref[0]

    @pl.when(first == 0)
    def _():
        out_ref[...] += contrib


def _routing_metadata(sizes, T, nt):
    W = nt + G - 1
    off = jnp.concatenate([jnp.zeros(1, jnp.int32), jnp.cumsum(sizes)])
    start_t = off[:-1] // BT
    end_t = jnp.where(sizes > 0, (off[1:] - 1) // BT, start_t - 1)
    tiles = jnp.maximum(end_t - start_t + 1, 0)
    cum = jnp.cumsum(tiles)
    wids = jnp.arange(W, dtype=jnp.int32)
    gid = jnp.searchsorted(cum, wids, side='right').astype(jnp.int32)
    gid_c = jnp.minimum(gid, G - 1)
    prev_cum = jnp.where(gid_c > 0, cum[gid_c - 1], 0)
    rid = start_t[gid_c] + (wids - prev_cum)
    valid = wids < cum[-1]
    rid = jnp.where(valid, rid, nt - 1).astype(jnp.int32)
    gcl = jnp.where(valid, gid_c, G - 1).astype(jnp.int32)
    st = jnp.where(valid, off[gcl], 0).astype(jnp.int32)
    en = jnp.where(valid, off[gcl + 1], 0).astype(jnp.int32)
    first = jnp.concatenate([jnp.ones(1, jnp.int32),
                             (rid[1:] != rid[:-1]).astype(jnp.int32)])
    return rid, gcl, first, st, en, off


@jax.jit
def kernel(x, base_W, bias, qweight0, qweight1, scales0, scales1, indices):
    T, D = x.shape
    NOUT = base_W.shape[0]
    SL = NOUT // 2
    nt = T // BT
    W = nt + G - 1
    nc = NOUT // BN
    nc_s = SL // BN

    onehot = (indices[:, None] == jnp.arange(G)[None, :]).astype(jnp.int32)
    sizes = jnp.sum(onehot, axis=0)
    rank = (jnp.cumsum(onehot, axis=0) - onehot)[jnp.arange(T), indices]
    rid, gcl, first, st, en, off = _routing_metadata(sizes, T, nt)
    pos = off[indices] + rank
    x_s = jnp.zeros_like(x).at[pos].set(x, unique_indices=True)

    s0r = scales0.reshape(G, nc_s, 1, BN)
    s1r = scales1.reshape(G, nc_s, 1, BN)
    bias_r = bias.reshape(nc, 1, BN)

    grid_spec = pltpu.PrefetchScalarGridSpec(
        num_scalar_prefetch=5,
        grid=(nc, W),
        in_specs=[
            pl.BlockSpec((T, D), lambda c, w, *s: (0, 0)),
            pl.BlockSpec((1, BN, D),
                         lambda c, w, rows, grps, *s: (
                             jnp.where(c < nc_s, grps[w], 0),
                             jnp.where(c < nc_s, c, 0), 0)),
            pl.BlockSpec((1, BN, D),
                         lambda c, w, rows, grps, *s: (
                             jnp.where(c >= nc_s, grps[w], 0),
                             jnp.where(c >= nc_s, c - nc_s, 0), 0)),
            pl.BlockSpec((1, 1, 1, BN),
                         lambda c, w, rows, grps, *s: (
                             jnp.where(c < nc_s, grps[w], 0),
                             jnp.where(c < nc_s, c, 0), 0, 0)),
            pl.BlockSpec((1, 1, 1, BN),
                         lambda c, w, rows, grps, *s: (
                             jnp.where(c >= nc_s, grps[w], 0),
                             jnp.where(c >= nc_s, c - nc_s, 0), 0, 0)),
            pl.BlockSpec((BN, D), lambda c, w, *s: (c, 0)),
            pl.BlockSpec((1, 1, BN), lambda c, w, *s: (c, 0, 0)),
        ],
        out_specs=pl.BlockSpec((BT, BN), lambda c, w, rows, *s: (rows[w], c)),
        scratch_shapes=[
            pltpu.VMEM((T, D), jnp.bfloat16),
            pltpu.VMEM((BN, D), jnp.bfloat16),
        ],
    )

    out_s = pl.pallas_call(
        _grouped_body,
        grid_spec=grid_spec,
        out_shape=jax.ShapeDtypeStruct((T, NOUT), jnp.float32),
        compiler_params=pltpu.CompilerParams(
            dimension_semantics=("arbitrary", "arbitrary")),
    )(rid, gcl, first, st, en,
      x_s, qweight0, qweight1, s0r, s1r, base_W, bias_r)

    return jnp.take(out_s, pos, axis=0)
